# baseline (speedup 1.0000x reference)
import jax
import jax.numpy as jnp
from jax import lax
from jax.experimental import pallas as pl
from jax.experimental.pallas import tpu as pltpu
from jax.experimental.pallas import tpu_sc as plsc

N = 10000
D = 128
B = 4
PASSES = 3
NT = 16
NC = 2
ROWS_PT = N // NT
EDGES = 640000
ET = EDGES // NT
GROUP = 125
GPC = 8
CHUNK = GROUP * GPC
NCHUNK = ET // CHUNK
NBUF = 3
STG = -(-ROWS_PT // GROUP)


def _sc_body(nodes, dst_i, src_i, out_nodes, snap, tab, rows, idx_d, idx_s,
             gsems, ssems):
    c = lax.axis_index("c")
    s = lax.axis_index("s")
    r0 = s * ROWS_PT

    def stage_rows(src_ref, src_base, dst_ref, dst_base):
        ds = [None] * STG
        for q in range(STG):
            sz = min(GROUP, ROWS_PT - q * GROUP)
            b = q % NBUF
            if q >= NBUF:
                ds[q - NBUF].wait()
            pltpu.sync_copy(src_ref.at[pl.ds(src_base + q * GROUP, sz)],
                            rows[b].at[pl.ds(0, sz)])
            ds[q] = pltpu.async_copy(
                rows[b].at[pl.ds(0, sz)],
                dst_ref.at[pl.ds(dst_base + q * GROUP, sz)],
                ssems[b])
        for q in range(max(0, STG - NBUF), STG):
            ds[q].wait()

    for task in range(2):
        g = c * 2 + task
        stage_rows(nodes.at[g], r0, tab, r0)
        plsc.subcore_barrier()
        for p in range(PASSES):
            stage_rows(tab, r0, snap, g * N + r0)
            plsc.subcore_barrier()

            def chunk_body(ck, _, g=g):
                pltpu.sync_copy(dst_i.at[g, s, ck], idx_d)
                pltpu.sync_copy(src_i.at[g, s, ck], idx_s)
                gds = [None] * GPC
                sds = [None] * GPC
                for j in range(GPC):
                    b = j % NBUF
                    if j >= NBUF:
                        sds[j - NBUF].wait()
                    gds[j] = pltpu.async_copy(
                        snap.at[idx_s.at[j]], rows[b], gsems[b])
                    if j >= 1:
                        pb = (j - 1) % NBUF
                        gds[j - 1].wait()
                        sds[j - 1] = pltpu.async_copy(
                            rows[pb], tab.at[idx_d.at[j - 1]], ssems[pb],
                            add=True)
                lb = (GPC - 1) % NBUF
                gds[GPC - 1].wait()
                sds[GPC - 1] = pltpu.async_copy(
                    rows[lb], tab.at[idx_d.at[GPC - 1]], ssems[lb], add=True)
                for j in range(GPC - NBUF, GPC):
                    sds[j].wait()
                return 0

            lax.fori_loop(0, NCHUNK, chunk_body, 0)
            plsc.subcore_barrier()
        stage_rows(tab, r0, out_nodes.at[g], r0)
        plsc.subcore_barrier()


def _message_passing(nodes, dst_i, src_i):
    mesh = plsc.VectorSubcoreMesh(core_axis_name="c", subcore_axis_name="s")
    k = pl.kernel(
        _sc_body,
        out_type=[
            jax.ShapeDtypeStruct((B, N, D), jnp.float32),
            jax.ShapeDtypeStruct((B * N, D), jnp.float32),
        ],
        mesh=mesh,
        scratch_types=[
            pltpu.VMEM_SHARED((N, D), jnp.float32),
            [pltpu.VMEM((GROUP, D), jnp.float32) for _ in range(NBUF)],
            pltpu.VMEM((GPC, GROUP), jnp.int32),
            pltpu.VMEM((GPC, GROUP), jnp.int32),
            [pltpu.SemaphoreType.DMA for _ in range(NBUF)],
            [pltpu.SemaphoreType.DMA for _ in range(NBUF)],
        ],
        compiler_params=pltpu.CompilerParams(use_tc_tiling_on_sc=False),
    )
    out, _ = k(nodes, dst_i, src_i)
    return out


def _pool_mlp_body(h_ref, pt_ref, w1a_ref, w1b_ref, b1_ref, w2_ref, b2_ref,
                   w3_ref, b3_ref, out_ref, acc):
    i = pl.program_id(0)
    ni = pl.num_programs(0)

    @pl.when(i == 0)
    def _():
        acc[...] = jnp.zeros_like(acc)

    acc[...] += jnp.sum(h_ref[...], axis=1)

    @pl.when(i == ni - 1)
    def _():
        v = jnp.log(acc[...])
        v = jnp.where(v != v, 0.0, v)
        v = jnp.maximum(v, 0.0)
        neg_inf = jnp.float32(-jnp.inf)
        m = jnp.max(jnp.where(jnp.isinf(v), neg_inf, v), axis=1,
                    keepdims=True)
        v = jnp.where(v == jnp.float32(jnp.inf), m, v)

        x = (jnp.dot(v, w1a_ref[...], preferred_element_type=jnp.float32)
             + pt_ref[...] * w1b_ref[...]
             + b1_ref[...])
        x = jnp.where(x >= 0, x, 0.01 * x)
        x = jnp.dot(x, w2_ref[...], preferred_element_type=jnp.float32) \
            + b2_ref[...]
        x = jnp.where(x >= 0, x, 0.01 * x)
        x = jnp.dot(x, w3_ref[...], preferred_element_type=jnp.float32) \
            + b3_ref[...]
        out_ref[...] = x


def _pool_mlp(final, pt, W1, b1, W2, b2, W3, b3):
    w1a = W1[:D]
    w1b = W1[D:D + 1]
    grid = 10
    rows = N // grid
    H = W1.shape[1]
    OUT = W3.shape[1]
    return pl.pallas_call(
        _pool_mlp_body,
        grid=(grid,),
        in_specs=[
            pl.BlockSpec((B, rows, D), lambda i: (0, i, 0)),
            pl.BlockSpec((B, 1), lambda i: (0, 0)),
            pl.BlockSpec((D, H), lambda i: (0, 0)),
            pl.BlockSpec((1, H), lambda i: (0, 0)),
            pl.BlockSpec((1, H), lambda i: (0, 0)),
            pl.BlockSpec((H, H), lambda i: (0, 0)),
            pl.BlockSpec((1, H), lambda i: (0, 0)),
            pl.BlockSpec((H, OUT), lambda i: (0, 0)),
            pl.BlockSpec((1, OUT), lambda i: (0, 0)),
        ],
        out_specs=pl.BlockSpec((B, OUT), lambda i: (0, 0)),
        out_shape=jax.ShapeDtypeStruct((B, OUT), jnp.float32),
        scratch_shapes=[
            pltpu.VMEM((B, D), jnp.float32),
        ],
    )(final, pt, w1a, w1b, b1.reshape(1, H), W2, b2.reshape(1, H), W3,
      b3.reshape(1, OUT))


def kernel(nodesBatch, backwards_edgeBatch, problemTypeBatch, W1, b1, W2, b2,
           W3, b3):
    e = backwards_edgeBatch.reshape(B, EDGES, 2)
    dst_i = e[..., 0].reshape(B, NT, NCHUNK, GPC, GROUP)
    g_off = (jnp.arange(B, dtype=jnp.int32) * N).reshape(B, 1, 1, 1, 1)
    src_i = e[..., 1].reshape(B, NT, NCHUNK, GPC, GROUP) + g_off

    final = _message_passing(nodesBatch, dst_i, src_i)
    return _pool_mlp(final, problemTypeBatch, W1, b1, W2, b2, W3, b3)


# speedup vs baseline: 1.1819x; 1.1819x over previous
import jax
import jax.numpy as jnp
from jax import lax
from jax.experimental import pallas as pl
from jax.experimental.pallas import tpu as pltpu
from jax.experimental.pallas import tpu_sc as plsc

N = 10000
D = 128
B = 4
PASSES = 3
NT = 16
NC = 2
ROWS_PT = N // NT
EDGES = 640000
ET = EDGES // NT
GROUP = 100
GPC = 25
CHUNK = GROUP * GPC
NCHUNK = ET // CHUNK
NBUF = 3
STG = -(-ROWS_PT // GROUP)


def _sc_body(nodes, dst_i, src_i, out_nodes, snap, tab, rows, idx_d, idx_s,
             gsems, ssems):
    c = lax.axis_index("c")
    s = lax.axis_index("s")
    r0 = s * ROWS_PT

    def stage_rows(src_ref, src_base, dst_ref, dst_base):
        ds = [None] * STG
        for q in range(STG):
            sz = min(GROUP, ROWS_PT - q * GROUP)
            b = q % NBUF
            if q >= NBUF:
                ds[q - NBUF].wait()
            pltpu.sync_copy(src_ref.at[pl.ds(src_base + q * GROUP, sz)],
                            rows[b].at[pl.ds(0, sz)])
            ds[q] = pltpu.async_copy(
                rows[b].at[pl.ds(0, sz)],
                dst_ref.at[pl.ds(dst_base + q * GROUP, sz)],
                ssems[b])
        for q in range(max(0, STG - NBUF), STG):
            ds[q].wait()

    for task in range(2):
        g = c * 2 + task
        stage_rows(nodes.at[g], r0, tab, r0)
        plsc.subcore_barrier()
        for p in range(PASSES):
            stage_rows(tab, r0, snap, g * N + r0)
            plsc.subcore_barrier()

            def chunk_body(ck, _, g=g):
                pltpu.sync_copy(dst_i.at[g, s, ck], idx_d)
                pltpu.sync_copy(src_i.at[g, s, ck], idx_s)
                gds = [None] * GPC
                sds = [None] * GPC
                for j in range(GPC):
                    b = j % NBUF
                    if j >= NBUF:
                        sds[j - NBUF].wait()
                    gds[j] = pltpu.async_copy(
                        snap.at[idx_s.at[j]], rows[b], gsems[b])
                    if j >= 1:
                        pb = (j - 1) % NBUF
                        gds[j - 1].wait()
                        sds[j - 1] = pltpu.async_copy(
                            rows[pb], tab.at[idx_d.at[j - 1]], ssems[pb],
                            add=True)
                lb = (GPC - 1) % NBUF
                gds[GPC - 1].wait()
                sds[GPC - 1] = pltpu.async_copy(
                    rows[lb], tab.at[idx_d.at[GPC - 1]], ssems[lb], add=True)
                for j in range(GPC - NBUF, GPC):
                    sds[j].wait()
                return 0

            lax.fori_loop(0, NCHUNK, chunk_body, 0)
            plsc.subcore_barrier()
        stage_rows(tab, r0, out_nodes.at[g], r0)
        plsc.subcore_barrier()


def _message_passing(nodes, dst_i, src_i):
    mesh = plsc.VectorSubcoreMesh(core_axis_name="c", subcore_axis_name="s")
    k = pl.kernel(
        _sc_body,
        out_type=[
            jax.ShapeDtypeStruct((B, N, D), jnp.float32),
            jax.ShapeDtypeStruct((B * N, D), jnp.float32),
        ],
        mesh=mesh,
        scratch_types=[
            pltpu.VMEM_SHARED((N, D), jnp.float32),
            [pltpu.VMEM((GROUP, D), jnp.float32) for _ in range(NBUF)],
            pltpu.VMEM((GPC, GROUP), jnp.int32),
            pltpu.VMEM((GPC, GROUP), jnp.int32),
            [pltpu.SemaphoreType.DMA for _ in range(NBUF)],
            [pltpu.SemaphoreType.DMA for _ in range(NBUF)],
        ],
        compiler_params=pltpu.CompilerParams(use_tc_tiling_on_sc=False),
    )
    out, _ = k(nodes, dst_i, src_i)
    return out


def _pool_mlp_body(h_ref, pt_ref, w1a_ref, w1b_ref, b1_ref, w2_ref, b2_ref,
                   w3_ref, b3_ref, out_ref, acc):
    i = pl.program_id(0)
    ni = pl.num_programs(0)

    @pl.when(i == 0)
    def _():
        acc[...] = jnp.zeros_like(acc)

    acc[...] += jnp.sum(h_ref[...], axis=1)

    @pl.when(i == ni - 1)
    def _():
        v = jnp.log(acc[...])
        v = jnp.where(v != v, 0.0, v)
        v = jnp.maximum(v, 0.0)
        neg_inf = jnp.float32(-jnp.inf)
        m = jnp.max(jnp.where(jnp.isinf(v), neg_inf, v), axis=1,
                    keepdims=True)
        v = jnp.where(v == jnp.float32(jnp.inf), m, v)

        x = (jnp.dot(v, w1a_ref[...], preferred_element_type=jnp.float32)
             + pt_ref[...] * w1b_ref[...]
             + b1_ref[...])
        x = jnp.where(x >= 0, x, 0.01 * x)
        x = jnp.dot(x, w2_ref[...], preferred_element_type=jnp.float32) \
            + b2_ref[...]
        x = jnp.where(x >= 0, x, 0.01 * x)
        x = jnp.dot(x, w3_ref[...], preferred_element_type=jnp.float32) \
            + b3_ref[...]
        out_ref[...] = x


def _pool_mlp(final, pt, W1, b1, W2, b2, W3, b3):
    w1a = W1[:D]
    w1b = W1[D:D + 1]
    grid = 10
    rows = N // grid
    H = W1.shape[1]
    OUT = W3.shape[1]
    return pl.pallas_call(
        _pool_mlp_body,
        grid=(grid,),
        in_specs=[
            pl.BlockSpec((B, rows, D), lambda i: (0, i, 0)),
            pl.BlockSpec((B, 1), lambda i: (0, 0)),
            pl.BlockSpec((D, H), lambda i: (0, 0)),
            pl.BlockSpec((1, H), lambda i: (0, 0)),
            pl.BlockSpec((1, H), lambda i: (0, 0)),
            pl.BlockSpec((H, H), lambda i: (0, 0)),
            pl.BlockSpec((1, H), lambda i: (0, 0)),
            pl.BlockSpec((H, OUT), lambda i: (0, 0)),
            pl.BlockSpec((1, OUT), lambda i: (0, 0)),
        ],
        out_specs=pl.BlockSpec((B, OUT), lambda i: (0, 0)),
        out_shape=jax.ShapeDtypeStruct((B, OUT), jnp.float32),
        scratch_shapes=[
            pltpu.VMEM((B, D), jnp.float32),
        ],
    )(final, pt, w1a, w1b, b1.reshape(1, H), W2, b2.reshape(1, H), W3,
      b3.reshape(1, OUT))


def kernel(nodesBatch, backwards_edgeBatch, problemTypeBatch, W1, b1, W2, b2,
           W3, b3):
    e = backwards_edgeBatch.reshape(B, EDGES, 2)
    dst_i = e[..., 0].reshape(B, NT, NCHUNK, GPC, GROUP)
    g_off = (jnp.arange(B, dtype=jnp.int32) * N).reshape(B, 1, 1, 1, 1)
    src_i = e[..., 1].reshape(B, NT, NCHUNK, GPC, GROUP) + g_off

    final = _message_passing(nodesBatch, dst_i, src_i)
    return _pool_mlp(final, problemTypeBatch, W1, b1, W2, b2, W3, b3)
